# baseline (speedup 1.0000x reference)
import functools

import jax
import jax.numpy as jnp
from jax import lax
from jax.experimental import pallas as pl
from jax.experimental.pallas import tpu as pltpu
from jax.experimental.pallas import tpu_sc as plsc

N_NODES = 10000
N_EDGES = 320000
D_FEAT = 128
D_EDGE = 16
D_HID = 128
D_OUT = 128

NC = 2
NS = 16
NGRP = 4
NFS = 4
FPT = D_EDGE // NFS
EDGES_PER_SC = N_EDGES // NC
EDGES_PER_GRP = EDGES_PER_SC // NGRP
CHUNK = 4000
NCHUNK = EDGES_PER_GRP // CHUNK
VSTEPS = CHUNK // 16
N_PAD = 10240


def _seg_sum_sc(edge_index, edge_attr_t):
    mesh = plsc.VectorSubcoreMesh(core_axis_name="c", subcore_axis_name="s")

    @functools.partial(
        pl.kernel,
        mesh=mesh,
        out_type=jax.ShapeDtypeStruct((NC, NGRP, D_EDGE, N_PAD), jnp.float32),
        scratch_types=[
            pltpu.VMEM((2, CHUNK), jnp.int32),
            pltpu.VMEM((2, FPT, CHUNK), jnp.float32),
            pltpu.VMEM((FPT, N_PAD), jnp.float32),
            pltpu.SemaphoreType.DMA,
            pltpu.SemaphoreType.DMA,
            pltpu.SemaphoreType.DMA,
            pltpu.SemaphoreType.DMA,
        ],
        compiler_params=pltpu.CompilerParams(
            use_tc_tiling_on_sc=False, needs_layout_passes=False),
    )
    def k(idx_hbm, eat_hbm, out_hbm, idx_v, val_v, agg_v,
          si0, si1, sv0, sv1):
        cid = lax.axis_index("c")
        sid = lax.axis_index("s")
        grp = sid % NGRP
        fq = sid // NGRP
        base = cid * EDGES_PER_SC + grp * EDGES_PER_GRP
        sis = (si0, si1)
        svs = (sv0, sv1)

        def start(j):
            b = j % 2
            off = base + j * CHUNK
            ci = pltpu.async_copy(
                idx_hbm.at[0, pl.ds(off, CHUNK)], idx_v.at[b], sis[b])
            cv = pltpu.async_copy(
                eat_hbm.at[pl.ds(fq * FPT, FPT), pl.ds(off, CHUNK)],
                val_v.at[b], svs[b])
            return ci, cv

        pend = start(0)

        zrow = jnp.zeros((16,), jnp.float32)

        def zb(i, carry):
            for f in range(FPT):
                agg_v[f, pl.ds(i * 16, 16)] = zrow
            return carry

        lax.fori_loop(0, N_PAD // 16, zb, 0)

        for j in range(NCHUNK):
            b = j % 2
            ci, cv = pend
            ci.wait()
            cv.wait()
            if j + 1 < NCHUNK:
                pend = start(j + 1)

            def step(i, carry):
                idxv = idx_v[b, pl.ds(i * 16, 16)]
                for f in range(FPT):
                    valv = val_v[b, f, pl.ds(i * 16, 16)]
                    plsc.addupdate_scatter(agg_v.at[f], [idxv], valv)
                return carry

            lax.fori_loop(0, VSTEPS, step, 0, unroll=10)

        pltpu.sync_copy(
            agg_v, out_hbm.at[cid, grp, pl.ds(fq * FPT, FPT)])

    return k(edge_index, edge_attr_t)


ROW_BLK = 1024
N_BLK = (N_NODES + ROW_BLK - 1) // ROW_BLK


def _mlp_body(x_r, xl_r, z_r, parts_r, wx_r, wl_r, wz_r, wa_r, b1_r,
              w2_r, b2_r, o_r):
    acc = jnp.dot(x_r[...], wx_r[...], preferred_element_type=jnp.float32)
    acc = acc + jnp.dot(xl_r[...], wl_r[...], preferred_element_type=jnp.float32)
    acc = acc + jnp.dot(z_r[...], wz_r[...], preferred_element_type=jnp.float32)
    p = parts_r[...]
    agg_t = p[0] + p[1]
    for i in range(2, NC * NGRP):
        agg_t = agg_t + p[i]
    acc = acc + lax.dot_general(
        agg_t, wa_r[...], (((0,), (0,)), ((), ())),
        preferred_element_type=jnp.float32)
    h = jnp.maximum(acc + b1_r[...], 0.0)
    o_r[...] = jnp.dot(h, w2_r[...], preferred_element_type=jnp.float32) + b2_r[...]


def _mlp_tc(x, xl, z, parts, wx, wl, wz, wa, b1, w2, b2):
    row_spec = pl.BlockSpec((ROW_BLK, D_FEAT), lambda i: (i, 0))
    parts_spec = pl.BlockSpec((NC * NGRP, D_EDGE, ROW_BLK), lambda i: (0, 0, i))

    def full(shape):
        return pl.BlockSpec(shape, lambda i: (0,) * len(shape))

    return pl.pallas_call(
        _mlp_body,
        grid=(N_BLK,),
        in_specs=[
            row_spec, row_spec, row_spec, parts_spec,
            full((D_FEAT, D_HID)), full((D_FEAT, D_HID)), full((D_FEAT, D_HID)),
            full((D_EDGE, D_HID)), full((1, D_HID)),
            full((D_HID, D_OUT)), full((1, D_OUT)),
        ],
        out_specs=pl.BlockSpec((ROW_BLK, D_OUT), lambda i: (i, 0)),
        out_shape=jax.ShapeDtypeStruct((N_NODES, D_OUT), jnp.float32),
        compiler_params=pltpu.CompilerParams(
            dimension_semantics=("arbitrary",),
        ),
    )(x, xl, z, parts, wx, wl, wz, wa, b1, w2, b2)


def kernel(x, x_lstm, encoded_z_gnss, edge_index, edge_attr, W1, b1, W2, b2):
    parts = _seg_sum_sc(edge_index.astype(jnp.int32), edge_attr.T)
    parts = parts.reshape(NC * NGRP, D_EDGE, N_PAD)
    wx = W1[0:D_FEAT]
    wl = W1[D_FEAT:2 * D_FEAT]
    wz = W1[2 * D_FEAT:3 * D_FEAT]
    wa = W1[3 * D_FEAT:]
    return _mlp_tc(
        x, x_lstm, encoded_z_gnss, parts,
        wx, wl, wz, wa, b1.reshape(1, D_HID), W2, b2.reshape(1, D_OUT),
    )


# speedup vs baseline: 1.3015x; 1.3015x over previous
import functools

import jax
import jax.numpy as jnp
from jax import lax
from jax.experimental import pallas as pl
from jax.experimental.pallas import tpu as pltpu
from jax.experimental.pallas import tpu_sc as plsc

N_NODES = 10000
N_EDGES = 320000
D_FEAT = 128
D_EDGE = 16
D_HID = 128
D_OUT = 128

NC = 2
NS = 16
NGRP = 4
NFS = 4
FPT = D_EDGE // NFS
EDGES_PER_SC = N_EDGES // NC
EDGES_PER_GRP = EDGES_PER_SC // NGRP
CHUNK = 4000
NCHUNK = EDGES_PER_GRP // CHUNK
VSTEPS = CHUNK // 16
N_PAD = 10240


def _seg_sum_sc(edge_index, edge_attr_t):
    mesh = plsc.VectorSubcoreMesh(core_axis_name="c", subcore_axis_name="s")

    @functools.partial(
        pl.kernel,
        mesh=mesh,
        out_type=jax.ShapeDtypeStruct((NC, NGRP, D_EDGE, N_PAD), jnp.float32),
        scratch_types=[
            pltpu.VMEM((2, CHUNK), jnp.int32),
            pltpu.VMEM((2, FPT, CHUNK), jnp.float32),
            pltpu.VMEM((FPT, N_PAD), jnp.float32),
            pltpu.SemaphoreType.DMA,
            pltpu.SemaphoreType.DMA,
            pltpu.SemaphoreType.DMA,
            pltpu.SemaphoreType.DMA,
        ],
        compiler_params=pltpu.CompilerParams(
            use_tc_tiling_on_sc=False, needs_layout_passes=False),
    )
    def k(idx_hbm, eat_hbm, out_hbm, idx_v, val_v, agg_v,
          si0, si1, sv0, sv1):
        cid = lax.axis_index("c")
        sid = lax.axis_index("s")
        grp = sid % NGRP
        fq = sid // NGRP
        base = cid * EDGES_PER_SC + grp * EDGES_PER_GRP
        sis = (si0, si1)
        svs = (sv0, sv1)

        def start(j):
            b = j % 2
            off = base + j * CHUNK
            ci = pltpu.async_copy(
                idx_hbm.at[0, pl.ds(off, CHUNK)], idx_v.at[b], sis[b])
            cv = pltpu.async_copy(
                eat_hbm.at[pl.ds(fq * FPT, FPT), pl.ds(off, CHUNK)],
                val_v.at[b], svs[b])
            return ci, cv

        pend = start(0)

        zrow = jnp.zeros((16,), jnp.float32)

        def zb(i, carry):
            for f in range(FPT):
                agg_v[f, pl.ds(i * 16, 16)] = zrow
            return carry

        lax.fori_loop(0, N_PAD // 16, zb, 0)

        for j in range(NCHUNK):
            b = j % 2
            ci, cv = pend
            ci.wait()
            cv.wait()
            if j + 1 < NCHUNK:
                pend = start(j + 1)

            @plsc.parallel_loop(0, VSTEPS, unroll=5)
            def step(i):
                idxv = idx_v[b, pl.ds(i * 16, 16)]
                for f in range(FPT):
                    valv = val_v[b, f, pl.ds(i * 16, 16)]
                    plsc.addupdate_scatter(agg_v.at[f], [idxv], valv)

        pltpu.sync_copy(
            agg_v, out_hbm.at[cid, grp, pl.ds(fq * FPT, FPT)])

    return k(edge_index, edge_attr_t)


ROW_BLK = 1024
N_BLK = (N_NODES + ROW_BLK - 1) // ROW_BLK


def _mlp_body(x_r, xl_r, z_r, parts_r, wx_r, wl_r, wz_r, wa_r, b1_r,
              w2_r, b2_r, o_r):
    acc = jnp.dot(x_r[...], wx_r[...], preferred_element_type=jnp.float32)
    acc = acc + jnp.dot(xl_r[...], wl_r[...], preferred_element_type=jnp.float32)
    acc = acc + jnp.dot(z_r[...], wz_r[...], preferred_element_type=jnp.float32)
    p = parts_r[...]
    agg_t = p[0] + p[1]
    for i in range(2, NC * NGRP):
        agg_t = agg_t + p[i]
    acc = acc + lax.dot_general(
        agg_t, wa_r[...], (((0,), (0,)), ((), ())),
        preferred_element_type=jnp.float32)
    h = jnp.maximum(acc + b1_r[...], 0.0)
    o_r[...] = jnp.dot(h, w2_r[...], preferred_element_type=jnp.float32) + b2_r[...]


def _mlp_tc(x, xl, z, parts, wx, wl, wz, wa, b1, w2, b2):
    row_spec = pl.BlockSpec((ROW_BLK, D_FEAT), lambda i: (i, 0))
    parts_spec = pl.BlockSpec((NC * NGRP, D_EDGE, ROW_BLK), lambda i: (0, 0, i))

    def full(shape):
        return pl.BlockSpec(shape, lambda i: (0,) * len(shape))

    return pl.pallas_call(
        _mlp_body,
        grid=(N_BLK,),
        in_specs=[
            row_spec, row_spec, row_spec, parts_spec,
            full((D_FEAT, D_HID)), full((D_FEAT, D_HID)), full((D_FEAT, D_HID)),
            full((D_EDGE, D_HID)), full((1, D_HID)),
            full((D_HID, D_OUT)), full((1, D_OUT)),
        ],
        out_specs=pl.BlockSpec((ROW_BLK, D_OUT), lambda i: (i, 0)),
        out_shape=jax.ShapeDtypeStruct((N_NODES, D_OUT), jnp.float32),
        compiler_params=pltpu.CompilerParams(
            dimension_semantics=("arbitrary",),
        ),
    )(x, xl, z, parts, wx, wl, wz, wa, b1, w2, b2)


def kernel(x, x_lstm, encoded_z_gnss, edge_index, edge_attr, W1, b1, W2, b2):
    parts = _seg_sum_sc(edge_index.astype(jnp.int32), edge_attr.T)
    parts = parts.reshape(NC * NGRP, D_EDGE, N_PAD)
    wx = W1[0:D_FEAT]
    wl = W1[D_FEAT:2 * D_FEAT]
    wz = W1[2 * D_FEAT:3 * D_FEAT]
    wa = W1[3 * D_FEAT:]
    return _mlp_tc(
        x, x_lstm, encoded_z_gnss, parts,
        wx, wl, wz, wa, b1.reshape(1, D_HID), W2, b2.reshape(1, D_OUT),
    )


# speedup vs baseline: 1.5039x; 1.1556x over previous
import functools

import jax
import jax.numpy as jnp
from jax import lax
from jax.experimental import pallas as pl
from jax.experimental.pallas import tpu as pltpu
from jax.experimental.pallas import tpu_sc as plsc

N_NODES = 10000
N_EDGES = 320000
D_FEAT = 128
D_EDGE = 16
D_HID = 128
D_OUT = 128

NC = 2
NS = 16
FH = D_EDGE // NC
NFS = 4
FPT = FH // NFS
NGRP = NS // NFS
CBLK = N_EDGES // 128
GBLK = CBLK // NGRP
CB = 25
NCHUNK = GBLK // CB
N_PAD = 10240
NPB = N_PAD // 128


def _seg_sum_sc(ei_v, ea_v4):
    mesh = plsc.VectorSubcoreMesh(core_axis_name="c", subcore_axis_name="s")

    @functools.partial(
        pl.kernel,
        mesh=mesh,
        out_type=jax.ShapeDtypeStruct((NGRP, NC, NPB, FH, 128), jnp.float32),
        scratch_types=[
            pltpu.VMEM((2, CB, 1, 128), jnp.int32),
            pltpu.VMEM((2, CB, FPT, 128), jnp.float32),
            pltpu.VMEM((FPT, NPB, 1, 128), jnp.float32),
            pltpu.SemaphoreType.DMA,
            pltpu.SemaphoreType.DMA,
            pltpu.SemaphoreType.DMA,
            pltpu.SemaphoreType.DMA,
        ],
        compiler_params=pltpu.CompilerParams(
            use_tc_tiling_on_sc=False, needs_layout_passes=False),
    )
    def k(idx_hbm, ea_hbm, out_hbm, idx_v, val_v, agg_v, si0, si1, sv0, sv1):
        cid = lax.axis_index("c")
        sid = lax.axis_index("s")
        fq = sid // NGRP
        grp = sid % NGRP
        cbase = grp * GBLK
        sis = (si0, si1)
        svs = (sv0, sv1)

        def start(j):
            b = j % 2
            c0 = cbase + j * CB
            ci = pltpu.async_copy(
                idx_hbm.at[pl.ds(c0, CB), pl.ds(0, 1)], idx_v.at[b], sis[b])
            cv = pltpu.async_copy(
                ea_hbm.at[cid, pl.ds(c0, CB), pl.ds(fq * FPT, FPT)],
                val_v.at[b], svs[b])
            return ci, cv

        pend = start(0)

        zrow = jnp.zeros((16,), jnp.float32)

        def zb(c, carry):
            for f in range(FPT):
                for l in range(8):
                    agg_v[f, c, 0, pl.ds(l * 16, 16)] = zrow
            return carry

        lax.fori_loop(0, NPB, zb, 0)

        zi = jnp.zeros((16,), jnp.int32)
        for j in range(NCHUNK):
            b = j % 2
            ci, cv = pend
            ci.wait()
            cv.wait()
            if j + 1 < NCHUNK:
                pend = start(j + 1)

            @plsc.parallel_loop(0, CB, unroll=2)
            def step(c):
                for l in range(8):
                    idxv = idx_v[b, c, 0, pl.ds(l * 16, 16)]
                    hi = idxv >> 7
                    lo = idxv & 127
                    for f in range(FPT):
                        valv = val_v[b, c, f, pl.ds(l * 16, 16)]
                        plsc.addupdate_scatter(
                            agg_v.at[f], [hi, zi, lo], valv)

        for f in range(FPT):
            pltpu.sync_copy(
                agg_v.at[f],
                out_hbm.at[grp, cid, pl.ds(0, NPB),
                           pl.ds(fq * FPT + f, 1)])

    return k(ei_v, ea_v4)


ROW_BLK = 1024
N_BLK = N_PAD // ROW_BLK


def _mlp_body(x_r, xl_r, z_r, parts_r, wx_r, wl_r, wz_r, wa_r, b1_r,
              w2_r, b2_r, o_r):
    acc = jnp.dot(x_r[...], wx_r[...], preferred_element_type=jnp.float32)
    acc = acc + jnp.dot(xl_r[...], wl_r[...], preferred_element_type=jnp.float32)
    acc = acc + jnp.dot(z_r[...], wz_r[...], preferred_element_type=jnp.float32)
    p = parts_r[...]
    agg_t = p[0] + p[1]
    for i in range(2, NGRP):
        agg_t = agg_t + p[i]
    acc = acc + lax.dot_general(
        agg_t, wa_r[...], (((0,), (0,)), ((), ())),
        preferred_element_type=jnp.float32)
    h = jnp.maximum(acc + b1_r[...], 0.0)
    o_r[...] = jnp.dot(h, w2_r[...], preferred_element_type=jnp.float32) + b2_r[...]


def _mlp_tc(x, xl, z, parts, wx, wl, wz, wa, b1, w2, b2):
    row_spec = pl.BlockSpec((ROW_BLK, D_FEAT), lambda i: (i, 0))
    parts_spec = pl.BlockSpec((NGRP, D_EDGE, ROW_BLK), lambda i: (0, 0, i))

    def full(shape):
        return pl.BlockSpec(shape, lambda i: (0,) * len(shape))

    return pl.pallas_call(
        _mlp_body,
        grid=(N_BLK,),
        in_specs=[
            row_spec, row_spec, row_spec, parts_spec,
            full((D_FEAT, D_HID)), full((D_FEAT, D_HID)), full((D_FEAT, D_HID)),
            full((D_EDGE, D_HID)), full((1, D_HID)),
            full((D_HID, D_OUT)), full((1, D_OUT)),
        ],
        out_specs=pl.BlockSpec((ROW_BLK, D_OUT), lambda i: (i, 0)),
        out_shape=jax.ShapeDtypeStruct((N_NODES, D_OUT), jnp.float32),
        compiler_params=pltpu.CompilerParams(
            dimension_semantics=("arbitrary",),
        ),
    )(x, xl, z, parts, wx, wl, wz, wa, b1, w2, b2)


def kernel(x, x_lstm, encoded_z_gnss, edge_index, edge_attr, W1, b1, W2, b2):
    ei = edge_index.astype(jnp.int32)
    ei_v = ei.reshape(2, CBLK, 128).transpose(1, 0, 2)
    ea_v4 = edge_attr.reshape(CBLK, 128, NC, FH).transpose(2, 0, 3, 1)
    parts5 = _seg_sum_sc(ei_v, ea_v4)
    parts = parts5.transpose(0, 1, 3, 2, 4).reshape(NGRP, D_EDGE, N_PAD)
    wx = W1[0:D_FEAT]
    wl = W1[D_FEAT:2 * D_FEAT]
    wz = W1[2 * D_FEAT:3 * D_FEAT]
    wa = W1[3 * D_FEAT:]
    return _mlp_tc(
        x, x_lstm, encoded_z_gnss, parts,
        wx, wl, wz, wa, b1.reshape(1, D_HID), W2, b2.reshape(1, D_OUT),
    )


# speedup vs baseline: 1.5093x; 1.0035x over previous
import functools

import jax
import jax.numpy as jnp
from jax import lax
from jax.experimental import pallas as pl
from jax.experimental.pallas import tpu as pltpu
from jax.experimental.pallas import tpu_sc as plsc

N_NODES = 10000
N_EDGES = 320000
D_FEAT = 128
D_EDGE = 16
D_HID = 128
D_OUT = 128

NC = 2
NS = 16
FH = D_EDGE // NC
NFS = 4
FPT = FH // NFS
NGRP = NS // NFS
CBLK = N_EDGES // 128
GBLK = CBLK // NGRP
CB = 25
NCHUNK = GBLK // CB
N_PAD = 10240
NPB = N_PAD // 128


def _seg_sum_sc(ei_v, ea_v4):
    mesh = plsc.VectorSubcoreMesh(core_axis_name="c", subcore_axis_name="s")

    @functools.partial(
        pl.kernel,
        mesh=mesh,
        out_type=jax.ShapeDtypeStruct((NGRP, NC, NPB, FH, 128), jnp.float32),
        scratch_types=[
            pltpu.VMEM((2, CB, 1, 128), jnp.int32),
            pltpu.VMEM((2, CB, FPT, 128), jnp.float32),
            pltpu.VMEM((FPT, NPB, 1, 128), jnp.float32),
            pltpu.SemaphoreType.DMA,
            pltpu.SemaphoreType.DMA,
            pltpu.SemaphoreType.DMA,
            pltpu.SemaphoreType.DMA,
        ],
        compiler_params=pltpu.CompilerParams(
            use_tc_tiling_on_sc=False, needs_layout_passes=False),
    )
    def k(idx_hbm, ea_hbm, out_hbm, idx_v, val_v, agg_v, si0, si1, sv0, sv1):
        cid = lax.axis_index("c")
        sid = lax.axis_index("s")
        fq = sid // NGRP
        grp = sid % NGRP
        cbase = grp * GBLK
        sis = (si0, si1)
        svs = (sv0, sv1)

        def start(j):
            b = j % 2
            c0 = cbase + j * CB
            ci = pltpu.async_copy(
                idx_hbm.at[pl.ds(c0, CB), pl.ds(0, 1)], idx_v.at[b], sis[b])
            cv = pltpu.async_copy(
                ea_hbm.at[cid, pl.ds(c0, CB), pl.ds(fq * FPT, FPT)],
                val_v.at[b], svs[b])
            return ci, cv

        pend = start(0)

        zrow = jnp.zeros((16,), jnp.float32)

        def zb(c, carry):
            for f in range(FPT):
                for l in range(8):
                    agg_v[f, c, 0, pl.ds(l * 16, 16)] = zrow
            return carry

        lax.fori_loop(0, NPB, zb, 0)

        zi = jnp.zeros((16,), jnp.int32)
        for j in range(NCHUNK):
            b = j % 2
            ci, cv = pend
            ci.wait()
            cv.wait()
            if j + 1 < NCHUNK:
                pend = start(j + 1)

            @plsc.parallel_loop(0, CB, unroll=3)
            def step(c):
                for l in range(8):
                    idxv = idx_v[b, c, 0, pl.ds(l * 16, 16)]
                    hi = idxv >> 7
                    lo = idxv & 127
                    for f in range(FPT):
                        valv = val_v[b, c, f, pl.ds(l * 16, 16)]
                        plsc.addupdate_scatter(
                            agg_v.at[f], [hi, zi, lo], valv)

        for f in range(FPT):
            pltpu.sync_copy(
                agg_v.at[f],
                out_hbm.at[grp, cid, pl.ds(0, NPB),
                           pl.ds(fq * FPT + f, 1)])

    return k(ei_v, ea_v4)


ROW_BLK = 1024
N_BLK = N_PAD // ROW_BLK


def _mlp_body(x_r, xl_r, z_r, parts_r, wx_r, wl_r, wz_r, wa_r, b1_r,
              w2_r, b2_r, o_r):
    acc = jnp.dot(x_r[...], wx_r[...], preferred_element_type=jnp.float32)
    acc = acc + jnp.dot(xl_r[...], wl_r[...], preferred_element_type=jnp.float32)
    acc = acc + jnp.dot(z_r[...], wz_r[...], preferred_element_type=jnp.float32)
    p = parts_r[...]
    agg_t = p[0] + p[1]
    for i in range(2, NGRP):
        agg_t = agg_t + p[i]
    acc = acc + lax.dot_general(
        agg_t, wa_r[...], (((0,), (0,)), ((), ())),
        preferred_element_type=jnp.float32)
    h = jnp.maximum(acc + b1_r[...], 0.0)
    o_r[...] = jnp.dot(h, w2_r[...], preferred_element_type=jnp.float32) + b2_r[...]


def _mlp_tc(x, xl, z, parts, wx, wl, wz, wa, b1, w2, b2):
    row_spec = pl.BlockSpec((ROW_BLK, D_FEAT), lambda i: (i, 0))
    parts_spec = pl.BlockSpec((NGRP, D_EDGE, ROW_BLK), lambda i: (0, 0, i))

    def full(shape):
        return pl.BlockSpec(shape, lambda i: (0,) * len(shape))

    return pl.pallas_call(
        _mlp_body,
        grid=(N_BLK,),
        in_specs=[
            row_spec, row_spec, row_spec, parts_spec,
            full((D_FEAT, D_HID)), full((D_FEAT, D_HID)), full((D_FEAT, D_HID)),
            full((D_EDGE, D_HID)), full((1, D_HID)),
            full((D_HID, D_OUT)), full((1, D_OUT)),
        ],
        out_specs=pl.BlockSpec((ROW_BLK, D_OUT), lambda i: (i, 0)),
        out_shape=jax.ShapeDtypeStruct((N_NODES, D_OUT), jnp.float32),
        compiler_params=pltpu.CompilerParams(
            dimension_semantics=("arbitrary",),
        ),
    )(x, xl, z, parts, wx, wl, wz, wa, b1, w2, b2)


def kernel(x, x_lstm, encoded_z_gnss, edge_index, edge_attr, W1, b1, W2, b2):
    ei = edge_index.astype(jnp.int32)
    ei_v = ei.reshape(2, CBLK, 128).transpose(1, 0, 2)
    ea_v4 = edge_attr.reshape(CBLK, 128, NC, FH).transpose(2, 0, 3, 1)
    parts5 = _seg_sum_sc(ei_v, ea_v4)
    parts = parts5.transpose(0, 1, 3, 2, 4).reshape(NGRP, D_EDGE, N_PAD)
    wx = W1[0:D_FEAT]
    wl = W1[D_FEAT:2 * D_FEAT]
    wz = W1[2 * D_FEAT:3 * D_FEAT]
    wa = W1[3 * D_FEAT:]
    return _mlp_tc(
        x, x_lstm, encoded_z_gnss, parts,
        wx, wl, wz, wa, b1.reshape(1, D_HID), W2, b2.reshape(1, D_OUT),
    )


# speedup vs baseline: 1.5640x; 1.0362x over previous
import functools

import jax
import jax.numpy as jnp
from jax import lax
from jax.experimental import pallas as pl
from jax.experimental.pallas import tpu as pltpu
from jax.experimental.pallas import tpu_sc as plsc

N_NODES = 10000
N_EDGES = 320000
D_FEAT = 128
D_EDGE = 16
D_HID = 128
D_OUT = 128

NC = 2
NS = 16
FH = D_EDGE // NC
NFS = 4
FPT = FH // NFS
NGRP = NS // NFS
CBLK = N_EDGES // 128
GBLK = CBLK // NGRP
CB = 25
NCHUNK = GBLK // CB
N_PAD = 10240
NPB = N_PAD // 128


def _seg_sum_sc(ei_v, ea_v4):
    mesh = plsc.VectorSubcoreMesh(core_axis_name="c", subcore_axis_name="s")

    @functools.partial(
        pl.kernel,
        mesh=mesh,
        out_type=jax.ShapeDtypeStruct((NGRP, NC, NPB, FH, 128), jnp.float32),
        scratch_types=[
            pltpu.VMEM((2, CB, 1, 128), jnp.int32),
            pltpu.VMEM((2, CB, FPT, 128), jnp.float32),
            pltpu.VMEM((FPT, NPB, 1, 128), jnp.float32),
            pltpu.SemaphoreType.DMA,
            pltpu.SemaphoreType.DMA,
            pltpu.SemaphoreType.DMA,
            pltpu.SemaphoreType.DMA,
        ],
        compiler_params=pltpu.CompilerParams(
            use_tc_tiling_on_sc=False, needs_layout_passes=False),
    )
    def k(idx_hbm, ea_hbm, out_hbm, idx_v, val_v, agg_v, si0, si1, sv0, sv1):
        cid = lax.axis_index("c")
        sid = lax.axis_index("s")
        fq = sid // NGRP
        grp = sid % NGRP
        cbase = grp * GBLK
        sis = (si0, si1)
        svs = (sv0, sv1)

        def start(j):
            b = j % 2
            c0 = cbase + j * CB
            ci = pltpu.async_copy(
                idx_hbm.at[pl.ds(c0, CB), pl.ds(0, 1)], idx_v.at[b], sis[b])
            cv = pltpu.async_copy(
                ea_hbm.at[cid, pl.ds(c0, CB), pl.ds(fq * FPT, FPT)],
                val_v.at[b], svs[b])
            return ci, cv

        pend = start(0)

        zrow = jnp.zeros((16,), jnp.float32)

        def zb(c, carry):
            for f in range(FPT):
                for l in range(8):
                    agg_v[f, c, 0, pl.ds(l * 16, 16)] = zrow
            return carry

        lax.fori_loop(0, NPB, zb, 0)

        zi = jnp.zeros((16,), jnp.int32)
        for j in range(NCHUNK):
            b = j % 2
            ci, cv = pend
            ci.wait()
            cv.wait()
            if j + 1 < NCHUNK:
                pend = start(j + 1)

            @plsc.parallel_loop(0, CB, unroll=3)
            def step(c):
                for l in range(8):
                    idxv = idx_v[b, c, 0, pl.ds(l * 16, 16)]
                    hi = idxv >> 7
                    lo = idxv & 127
                    for f in range(FPT):
                        valv = val_v[b, c, f, pl.ds(l * 16, 16)]
                        plsc.addupdate_scatter(
                            agg_v.at[f], [hi, zi, lo], valv)

        for f in range(FPT):
            pltpu.sync_copy(
                agg_v.at[f],
                out_hbm.at[grp, cid, pl.ds(0, NPB),
                           pl.ds(fq * FPT + f, 1)])

    return k(ei_v, ea_v4)


ROW_BLK = 1024
N_BLK = N_PAD // ROW_BLK


def _row_spec():
    return pl.BlockSpec((ROW_BLK, D_FEAT), lambda i: (i, 0))


def _full(shape):
    return pl.BlockSpec(shape, lambda i: (0,) * len(shape))


def _dense_body(x_r, xl_r, z_r, wx_r, wl_r, wz_r, b1_r, o_r):
    acc = jnp.dot(x_r[...], wx_r[...], preferred_element_type=jnp.float32)
    acc = acc + jnp.dot(xl_r[...], wl_r[...], preferred_element_type=jnp.float32)
    acc = acc + jnp.dot(z_r[...], wz_r[...], preferred_element_type=jnp.float32)
    o_r[...] = acc + b1_r[...]


def _mlp_dense(x, xl, z, wx, wl, wz, b1):
    return pl.pallas_call(
        _dense_body,
        grid=(N_BLK,),
        in_specs=[
            _row_spec(), _row_spec(), _row_spec(),
            _full((D_FEAT, D_HID)), _full((D_FEAT, D_HID)),
            _full((D_FEAT, D_HID)), _full((1, D_HID)),
        ],
        out_specs=pl.BlockSpec((ROW_BLK, D_HID), lambda i: (i, 0)),
        out_shape=jax.ShapeDtypeStruct((N_NODES, D_HID), jnp.float32),
        compiler_params=pltpu.CompilerParams(
            dimension_semantics=("arbitrary",),
        ),
    )(x, xl, z, wx, wl, wz, b1)


def _final_body(acc_r, parts_r, wa_r, w2_r, b2_r, o_r):
    p = parts_r[...]
    agg_t = p[0] + p[1]
    for i in range(2, NGRP):
        agg_t = agg_t + p[i]
    acc = acc_r[...] + lax.dot_general(
        agg_t, wa_r[...], (((0,), (0,)), ((), ())),
        preferred_element_type=jnp.float32)
    h = jnp.maximum(acc, 0.0)
    o_r[...] = jnp.dot(h, w2_r[...], preferred_element_type=jnp.float32) + b2_r[...]


def _mlp_final(acc, parts, wa, w2, b2):
    parts_spec = pl.BlockSpec((NGRP, D_EDGE, ROW_BLK), lambda i: (0, 0, i))
    return pl.pallas_call(
        _final_body,
        grid=(N_BLK,),
        in_specs=[
            pl.BlockSpec((ROW_BLK, D_HID), lambda i: (i, 0)), parts_spec,
            _full((D_EDGE, D_HID)), _full((D_HID, D_OUT)), _full((1, D_OUT)),
        ],
        out_specs=pl.BlockSpec((ROW_BLK, D_OUT), lambda i: (i, 0)),
        out_shape=jax.ShapeDtypeStruct((N_NODES, D_OUT), jnp.float32),
        compiler_params=pltpu.CompilerParams(
            dimension_semantics=("arbitrary",),
        ),
    )(acc, parts, wa, w2, b2)


def kernel(x, x_lstm, encoded_z_gnss, edge_index, edge_attr, W1, b1, W2, b2):
    ei = edge_index.astype(jnp.int32)
    ei_v = ei.reshape(2, CBLK, 128).transpose(1, 0, 2)
    ea_v4 = edge_attr.reshape(CBLK, 128, NC, FH).transpose(2, 0, 3, 1)
    parts5 = _seg_sum_sc(ei_v, ea_v4)
    parts = parts5.transpose(0, 1, 3, 2, 4).reshape(NGRP, D_EDGE, N_PAD)
    wx = W1[0:D_FEAT]
    wl = W1[D_FEAT:2 * D_FEAT]
    wz = W1[2 * D_FEAT:3 * D_FEAT]
    wa = W1[3 * D_FEAT:]
    acc = _mlp_dense(x, x_lstm, encoded_z_gnss, wx, wl, wz,
                     b1.reshape(1, D_HID))
    return _mlp_final(acc, parts, wa, W2, b2.reshape(1, D_OUT))


# speedup vs baseline: 1.7629x; 1.1272x over previous
import functools

import jax
import jax.numpy as jnp
from jax import lax
from jax.experimental import pallas as pl
from jax.experimental.pallas import tpu as pltpu
from jax.experimental.pallas import tpu_sc as plsc

N_NODES = 10000
N_EDGES = 320000
D_FEAT = 128
D_EDGE = 16
D_HID = 128
D_OUT = 128

NC = 2
NS = 16
FH = D_EDGE // NC
NFS = 4
FPT = FH // NFS
NGRP = NS // NFS
CBLK = N_EDGES // 128
GBLK = CBLK // NGRP
CB = 125
NCHUNK = GBLK // CB
N_PAD = 10240
NPB = N_PAD // 128


def _seg_sum_sc(ei_v, ea_v4):
    mesh = plsc.VectorSubcoreMesh(core_axis_name="c", subcore_axis_name="s")

    @functools.partial(
        pl.kernel,
        mesh=mesh,
        out_type=jax.ShapeDtypeStruct((NGRP, NC, NPB, FH, 128), jnp.float32),
        scratch_types=[
            pltpu.VMEM((2, CB, 1, 128), jnp.int32),
            pltpu.VMEM((2, CB, FPT, 128), jnp.float32),
            pltpu.VMEM((FPT, NPB, 1, 128), jnp.float32),
            pltpu.SemaphoreType.DMA,
            pltpu.SemaphoreType.DMA,
            pltpu.SemaphoreType.DMA,
            pltpu.SemaphoreType.DMA,
        ],
        compiler_params=pltpu.CompilerParams(
            use_tc_tiling_on_sc=False, needs_layout_passes=False),
    )
    def k(idx_hbm, ea_hbm, out_hbm, idx_v, val_v, agg_v, si0, si1, sv0, sv1):
        cid = lax.axis_index("c")
        sid = lax.axis_index("s")
        fq = sid // NGRP
        grp = sid % NGRP
        cbase = grp * GBLK
        sis = (si0, si1)
        svs = (sv0, sv1)

        def start(j):
            b = j % 2
            c0 = cbase + j * CB
            ci = pltpu.async_copy(
                idx_hbm.at[pl.ds(c0, CB), pl.ds(0, 1)], idx_v.at[b], sis[b])
            cv = pltpu.async_copy(
                ea_hbm.at[cid, pl.ds(c0, CB), pl.ds(fq * FPT, FPT)],
                val_v.at[b], svs[b])
            return ci, cv

        pend = start(0)

        zrow = jnp.zeros((16,), jnp.float32)

        def zb(c, carry):
            for f in range(FPT):
                for l in range(8):
                    agg_v[f, c, 0, pl.ds(l * 16, 16)] = zrow
            return carry

        lax.fori_loop(0, NPB, zb, 0)

        zi = jnp.zeros((16,), jnp.int32)
        for j in range(NCHUNK):
            b = j % 2
            ci, cv = pend
            ci.wait()
            cv.wait()
            if j + 1 < NCHUNK:
                pend = start(j + 1)

            @plsc.parallel_loop(0, CB, unroll=3)
            def step(c):
                for l in range(8):
                    idxv = idx_v[b, c, 0, pl.ds(l * 16, 16)]
                    hi = idxv >> 7
                    lo = idxv & 127
                    for f in range(FPT):
                        valv = val_v[b, c, f, pl.ds(l * 16, 16)]
                        plsc.addupdate_scatter(
                            agg_v.at[f], [hi, zi, lo], valv)

        for f in range(FPT):
            pltpu.sync_copy(
                agg_v.at[f],
                out_hbm.at[grp, cid, pl.ds(0, NPB),
                           pl.ds(fq * FPT + f, 1)])

    return k(ei_v, ea_v4)


ROW_BLK = 1024
N_BLK = N_PAD // ROW_BLK


def _row_spec():
    return pl.BlockSpec((ROW_BLK, D_FEAT), lambda i: (i, 0))


def _full(shape):
    return pl.BlockSpec(shape, lambda i: (0,) * len(shape))


def _dense_body(x_r, xl_r, z_r, wx_r, wl_r, wz_r, b1_r, o_r):
    acc = jnp.dot(x_r[...], wx_r[...], preferred_element_type=jnp.float32)
    acc = acc + jnp.dot(xl_r[...], wl_r[...], preferred_element_type=jnp.float32)
    acc = acc + jnp.dot(z_r[...], wz_r[...], preferred_element_type=jnp.float32)
    o_r[...] = acc + b1_r[...]


def _mlp_dense(x, xl, z, wx, wl, wz, b1):
    return pl.pallas_call(
        _dense_body,
        grid=(N_BLK,),
        in_specs=[
            _row_spec(), _row_spec(), _row_spec(),
            _full((D_FEAT, D_HID)), _full((D_FEAT, D_HID)),
            _full((D_FEAT, D_HID)), _full((1, D_HID)),
        ],
        out_specs=pl.BlockSpec((ROW_BLK, D_HID), lambda i: (i, 0)),
        out_shape=jax.ShapeDtypeStruct((N_NODES, D_HID), jnp.float32),
        compiler_params=pltpu.CompilerParams(
            dimension_semantics=("arbitrary",),
        ),
    )(x, xl, z, wx, wl, wz, b1)


def _final_body(acc_r, parts_r, wa_r, w2_r, b2_r, o_r):
    p = parts_r[...]
    agg_t = p[0] + p[1]
    for i in range(2, NGRP):
        agg_t = agg_t + p[i]
    acc = acc_r[...] + lax.dot_general(
        agg_t, wa_r[...], (((0,), (0,)), ((), ())),
        preferred_element_type=jnp.float32)
    h = jnp.maximum(acc, 0.0)
    o_r[...] = jnp.dot(h, w2_r[...], preferred_element_type=jnp.float32) + b2_r[...]


def _mlp_final(acc, parts, wa, w2, b2):
    parts_spec = pl.BlockSpec((NGRP, D_EDGE, ROW_BLK), lambda i: (0, 0, i))
    return pl.pallas_call(
        _final_body,
        grid=(N_BLK,),
        in_specs=[
            pl.BlockSpec((ROW_BLK, D_HID), lambda i: (i, 0)), parts_spec,
            _full((D_EDGE, D_HID)), _full((D_HID, D_OUT)), _full((1, D_OUT)),
        ],
        out_specs=pl.BlockSpec((ROW_BLK, D_OUT), lambda i: (i, 0)),
        out_shape=jax.ShapeDtypeStruct((N_NODES, D_OUT), jnp.float32),
        compiler_params=pltpu.CompilerParams(
            dimension_semantics=("arbitrary",),
        ),
    )(acc, parts, wa, w2, b2)


def kernel(x, x_lstm, encoded_z_gnss, edge_index, edge_attr, W1, b1, W2, b2):
    ei = edge_index.astype(jnp.int32)
    ei_v = ei.reshape(2, CBLK, 128).transpose(1, 0, 2)
    ea_v4 = edge_attr.reshape(CBLK, 128, NC, FH).transpose(2, 0, 3, 1)
    parts5 = _seg_sum_sc(ei_v, ea_v4)
    parts = parts5.transpose(0, 1, 3, 2, 4).reshape(NGRP, D_EDGE, N_PAD)
    wx = W1[0:D_FEAT]
    wl = W1[D_FEAT:2 * D_FEAT]
    wz = W1[2 * D_FEAT:3 * D_FEAT]
    wa = W1[3 * D_FEAT:]
    acc = _mlp_dense(x, x_lstm, encoded_z_gnss, wx, wl, wz,
                     b1.reshape(1, D_HID))
    return _mlp_final(acc, parts, wa, W2, b2.reshape(1, D_OUT))


# speedup vs baseline: 1.7829x; 1.0114x over previous
import functools

import jax
import jax.numpy as jnp
from jax import lax
from jax.experimental import pallas as pl
from jax.experimental.pallas import tpu as pltpu
from jax.experimental.pallas import tpu_sc as plsc

N_NODES = 10000
N_EDGES = 320000
D_FEAT = 128
D_EDGE = 16
D_HID = 128
D_OUT = 128

NC = 2
NS = 16
FH = D_EDGE // NC
NFS = 4
FPT = FH // NFS
NGRP = NS // NFS
CBLK = N_EDGES // 128
GBLK = CBLK // NGRP
CB = 125
NCHUNK = GBLK // CB
N_PAD = 10240
NPB = N_PAD // 128


def _seg_sum_sc(ei_v, ea_v4):
    mesh = plsc.VectorSubcoreMesh(core_axis_name="c", subcore_axis_name="s")

    @functools.partial(
        pl.kernel,
        mesh=mesh,
        out_type=jax.ShapeDtypeStruct((NGRP, NC, NPB, FH, 128), jnp.float32),
        scratch_types=[
            pltpu.VMEM((2, CB, 1, 128), jnp.int32),
            pltpu.VMEM((2, CB, FPT, 128), jnp.float32),
            pltpu.VMEM((FPT, NPB, 1, 128), jnp.float32),
            pltpu.SemaphoreType.DMA,
            pltpu.SemaphoreType.DMA,
            pltpu.SemaphoreType.DMA,
            pltpu.SemaphoreType.DMA,
        ],
        compiler_params=pltpu.CompilerParams(
            use_tc_tiling_on_sc=False, needs_layout_passes=False),
    )
    def k(idx_hbm, ea_hbm, out_hbm, idx_v, val_v, agg_v, si0, si1, sv0, sv1):
        cid = lax.axis_index("c")
        sid = lax.axis_index("s")
        fq = sid // NGRP
        grp = sid % NGRP
        cbase = grp * GBLK
        sis = (si0, si1)
        svs = (sv0, sv1)

        def start(j):
            b = j % 2
            c0 = cbase + j * CB
            ci = pltpu.async_copy(
                idx_hbm.at[pl.ds(c0, CB), pl.ds(0, 1)], idx_v.at[b], sis[b])
            cv = pltpu.async_copy(
                ea_hbm.at[cid, pl.ds(c0, CB), pl.ds(fq * FPT, FPT)],
                val_v.at[b], svs[b])
            return ci, cv

        pend = start(0)

        zrow = jnp.zeros((16,), jnp.float32)

        def zb(c, carry):
            for f in range(FPT):
                for l in range(8):
                    agg_v[f, c, 0, pl.ds(l * 16, 16)] = zrow
            return carry

        lax.fori_loop(0, NPB, zb, 0)

        zi = jnp.zeros((16,), jnp.int32)
        for j in range(NCHUNK):
            b = j % 2
            ci, cv = pend
            ci.wait()
            cv.wait()
            if j + 1 < NCHUNK:
                pend = start(j + 1)

            @plsc.parallel_loop(0, CB, unroll=3)
            def step(c):
                for l in range(8):
                    idxv = idx_v[b, c, 0, pl.ds(l * 16, 16)]
                    hi = idxv >> 7
                    lo = idxv & 127
                    for f in range(FPT):
                        valv = val_v[b, c, f, pl.ds(l * 16, 16)]
                        plsc.addupdate_scatter(
                            agg_v.at[f], [hi, zi, lo], valv)

        for f in range(FPT):
            pltpu.sync_copy(
                agg_v.at[f],
                out_hbm.at[grp, cid, pl.ds(0, NPB),
                           pl.ds(fq * FPT + f, 1)])

    return k(ei_v, ea_v4)


ROW_BLK = 1024
N_BLK = N_PAD // ROW_BLK


def _row_spec():
    return pl.BlockSpec((ROW_BLK, D_FEAT), lambda i: (i, 0))


def _full(shape):
    return pl.BlockSpec(shape, lambda i: (0,) * len(shape))


def _dense_body(x_r, xl_r, z_r, wx_r, wl_r, wz_r, b1_r, o_r):
    acc = jnp.dot(x_r[...], wx_r[...], preferred_element_type=jnp.float32)
    acc = acc + jnp.dot(xl_r[...], wl_r[...], preferred_element_type=jnp.float32)
    acc = acc + jnp.dot(z_r[...], wz_r[...], preferred_element_type=jnp.float32)
    o_r[...] = (acc + b1_r[...]).astype(jnp.bfloat16)


def _mlp_dense(x, xl, z, wx, wl, wz, b1):
    return pl.pallas_call(
        _dense_body,
        grid=(N_BLK,),
        in_specs=[
            _row_spec(), _row_spec(), _row_spec(),
            _full((D_FEAT, D_HID)), _full((D_FEAT, D_HID)),
            _full((D_FEAT, D_HID)), _full((1, D_HID)),
        ],
        out_specs=pl.BlockSpec((ROW_BLK, D_HID), lambda i: (i, 0)),
        out_shape=jax.ShapeDtypeStruct((N_NODES, D_HID), jnp.bfloat16),
        compiler_params=pltpu.CompilerParams(
            dimension_semantics=("arbitrary",),
        ),
    )(x, xl, z, wx, wl, wz, b1)


def _final_body(acc_r, parts_r, wa_r, w2_r, b2_r, o_r):
    p = parts_r[...]
    agg_t = p[0] + p[1]
    for i in range(2, NGRP):
        agg_t = agg_t + p[i]
    acc = acc_r[...].astype(jnp.float32) + lax.dot_general(
        agg_t, wa_r[...], (((0,), (0,)), ((), ())),
        preferred_element_type=jnp.float32)
    h = jnp.maximum(acc, 0.0)
    o_r[...] = jnp.dot(h, w2_r[...], preferred_element_type=jnp.float32) + b2_r[...]


def _mlp_final(acc, parts, wa, w2, b2):
    parts_spec = pl.BlockSpec((NGRP, D_EDGE, ROW_BLK), lambda i: (0, 0, i))
    return pl.pallas_call(
        _final_body,
        grid=(N_BLK,),
        in_specs=[
            pl.BlockSpec((ROW_BLK, D_HID), lambda i: (i, 0)), parts_spec,
            _full((D_EDGE, D_HID)), _full((D_HID, D_OUT)), _full((1, D_OUT)),
        ],
        out_specs=pl.BlockSpec((ROW_BLK, D_OUT), lambda i: (i, 0)),
        out_shape=jax.ShapeDtypeStruct((N_NODES, D_OUT), jnp.float32),
        compiler_params=pltpu.CompilerParams(
            dimension_semantics=("arbitrary",),
        ),
    )(acc, parts, wa, w2, b2)


def kernel(x, x_lstm, encoded_z_gnss, edge_index, edge_attr, W1, b1, W2, b2):
    ei = edge_index.astype(jnp.int32)
    ei_v = ei.reshape(2, CBLK, 128).transpose(1, 0, 2)
    ea_v4 = edge_attr.reshape(CBLK, 128, NC, FH).transpose(2, 0, 3, 1)
    parts5 = _seg_sum_sc(ei_v, ea_v4)
    parts = parts5.transpose(0, 1, 3, 2, 4).reshape(NGRP, D_EDGE, N_PAD)
    wx = W1[0:D_FEAT]
    wl = W1[D_FEAT:2 * D_FEAT]
    wz = W1[2 * D_FEAT:3 * D_FEAT]
    wa = W1[3 * D_FEAT:]
    acc = _mlp_dense(x, x_lstm, encoded_z_gnss, wx, wl, wz,
                     b1.reshape(1, D_HID))
    return _mlp_final(acc, parts, wa, W2, b2.reshape(1, D_OUT))


# speedup vs baseline: 1.8574x; 1.0418x over previous
import functools

import jax
import jax.numpy as jnp
from jax import lax
from jax.experimental import pallas as pl
from jax.experimental.pallas import tpu as pltpu
from jax.experimental.pallas import tpu_sc as plsc

N_NODES = 10000
N_EDGES = 320000
D_FEAT = 128
D_EDGE = 16
D_HID = 128
D_OUT = 128

NC = 2
NS = 16
FH = D_EDGE // NC
NFS = 4
FPT = FH // NFS
NGRP = NS // NFS
CBLK = N_EDGES // 128
GBLK = CBLK // NGRP
CB = 125
NCHUNK = GBLK // CB
N_PAD = 10240
NPB = N_PAD // 128


def _seg_sum_sc(ei_v, ea_v4):
    mesh = plsc.VectorSubcoreMesh(core_axis_name="c", subcore_axis_name="s")

    @functools.partial(
        pl.kernel,
        mesh=mesh,
        out_type=jax.ShapeDtypeStruct((NGRP, NC, NPB, FH, 128), jnp.float32),
        scratch_types=[
            pltpu.VMEM((2, CB, 1, 128), jnp.int32),
            pltpu.VMEM((2, CB, FPT, 128), jnp.float32),
            pltpu.VMEM((FPT, NPB, 1, 128), jnp.float32),
            pltpu.SemaphoreType.DMA,
            pltpu.SemaphoreType.DMA,
            pltpu.SemaphoreType.DMA,
            pltpu.SemaphoreType.DMA,
        ],
        compiler_params=pltpu.CompilerParams(
            use_tc_tiling_on_sc=False, needs_layout_passes=False),
    )
    def k(idx_hbm, ea_hbm, out_hbm, idx_v, val_v, agg_v, si0, si1, sv0, sv1):
        cid = lax.axis_index("c")
        sid = lax.axis_index("s")
        fq = sid // NGRP
        grp = sid % NGRP
        cbase = grp * GBLK
        sis = (si0, si1)
        svs = (sv0, sv1)

        def start(j):
            b = j % 2
            c0 = cbase + j * CB
            ci = pltpu.async_copy(
                idx_hbm.at[pl.ds(c0, CB), pl.ds(0, 1)], idx_v.at[b], sis[b])
            cv = pltpu.async_copy(
                ea_hbm.at[cid, pl.ds(c0, CB), pl.ds(fq * FPT, FPT)],
                val_v.at[b], svs[b])
            return ci, cv

        pend = start(0)

        zrow = jnp.zeros((16,), jnp.float32)

        def zb(c, carry):
            for f in range(FPT):
                for l in range(8):
                    agg_v[f, c, 0, pl.ds(l * 16, 16)] = zrow
            return carry

        lax.fori_loop(0, NPB, zb, 0)

        zi = jnp.zeros((16,), jnp.int32)
        for j in range(NCHUNK):
            b = j % 2
            ci, cv = pend
            ci.wait()
            cv.wait()
            if j + 1 < NCHUNK:
                pend = start(j + 1)

            @plsc.parallel_loop(0, CB, unroll=3)
            def step(c):
                for l in range(8):
                    idxv = idx_v[b, c, 0, pl.ds(l * 16, 16)]
                    hi = idxv >> 7
                    lo = idxv & 127
                    for f in range(FPT):
                        valv = val_v[b, c, f, pl.ds(l * 16, 16)]
                        plsc.addupdate_scatter(
                            agg_v.at[f], [hi, zi, lo], valv)

        for f in range(FPT):
            pltpu.sync_copy(
                agg_v.at[f],
                out_hbm.at[grp, cid, pl.ds(0, NPB),
                           pl.ds(fq * FPT + f, 1)])

    return k(ei_v, ea_v4)


ROW_BLK = 2048
N_BLK = N_PAD // ROW_BLK


def _row_spec():
    return pl.BlockSpec((ROW_BLK, D_FEAT), lambda i: (i, 0))


def _full(shape):
    return pl.BlockSpec(shape, lambda i: (0,) * len(shape))


def _dense_body(x_r, xl_r, z_r, wx_r, wl_r, wz_r, b1_r, o_r):
    acc = jnp.dot(x_r[...], wx_r[...], preferred_element_type=jnp.float32)
    acc = acc + jnp.dot(xl_r[...], wl_r[...], preferred_element_type=jnp.float32)
    acc = acc + jnp.dot(z_r[...], wz_r[...], preferred_element_type=jnp.float32)
    o_r[...] = (acc + b1_r[...]).astype(jnp.bfloat16)


def _mlp_dense(x, xl, z, wx, wl, wz, b1):
    return pl.pallas_call(
        _dense_body,
        grid=(N_BLK,),
        in_specs=[
            _row_spec(), _row_spec(), _row_spec(),
            _full((D_FEAT, D_HID)), _full((D_FEAT, D_HID)),
            _full((D_FEAT, D_HID)), _full((1, D_HID)),
        ],
        out_specs=pl.BlockSpec((ROW_BLK, D_HID), lambda i: (i, 0)),
        out_shape=jax.ShapeDtypeStruct((N_NODES, D_HID), jnp.bfloat16),
        compiler_params=pltpu.CompilerParams(
            dimension_semantics=("arbitrary",),
        ),
    )(x, xl, z, wx, wl, wz, b1)


def _final_body(acc_r, parts_r, wa_r, w2_r, b2_r, o_r):
    p = parts_r[...]
    agg_t = p[0] + p[1]
    for i in range(2, NGRP):
        agg_t = agg_t + p[i]
    acc = acc_r[...].astype(jnp.float32) + lax.dot_general(
        agg_t, wa_r[...], (((0,), (0,)), ((), ())),
        preferred_element_type=jnp.float32)
    h = jnp.maximum(acc, 0.0)
    o_r[...] = jnp.dot(h, w2_r[...], preferred_element_type=jnp.float32) + b2_r[...]


def _mlp_final(acc, parts, wa, w2, b2):
    parts_spec = pl.BlockSpec((NGRP, D_EDGE, ROW_BLK), lambda i: (0, 0, i))
    return pl.pallas_call(
        _final_body,
        grid=(N_BLK,),
        in_specs=[
            pl.BlockSpec((ROW_BLK, D_HID), lambda i: (i, 0)), parts_spec,
            _full((D_EDGE, D_HID)), _full((D_HID, D_OUT)), _full((1, D_OUT)),
        ],
        out_specs=pl.BlockSpec((ROW_BLK, D_OUT), lambda i: (i, 0)),
        out_shape=jax.ShapeDtypeStruct((N_NODES, D_OUT), jnp.float32),
        compiler_params=pltpu.CompilerParams(
            dimension_semantics=("arbitrary",),
        ),
    )(acc, parts, wa, w2, b2)


def kernel(x, x_lstm, encoded_z_gnss, edge_index, edge_attr, W1, b1, W2, b2):
    ei = edge_index.astype(jnp.int32)
    ei_v = ei.reshape(2, CBLK, 128).transpose(1, 0, 2)
    ea_v4 = edge_attr.reshape(CBLK, 128, NC, FH).transpose(2, 0, 3, 1)
    parts5 = _seg_sum_sc(ei_v, ea_v4)
    parts = parts5.transpose(0, 1, 3, 2, 4).reshape(NGRP, D_EDGE, N_PAD)
    wx = W1[0:D_FEAT]
    wl = W1[D_FEAT:2 * D_FEAT]
    wz = W1[2 * D_FEAT:3 * D_FEAT]
    wa = W1[3 * D_FEAT:]
    acc = _mlp_dense(x, x_lstm, encoded_z_gnss, wx, wl, wz,
                     b1.reshape(1, D_HID))
    return _mlp_final(acc, parts, wa, W2, b2.reshape(1, D_OUT))


# speedup vs baseline: 1.8951x; 1.0203x over previous
import functools

import jax
import jax.numpy as jnp
from jax import lax
from jax.experimental import pallas as pl
from jax.experimental.pallas import tpu as pltpu
from jax.experimental.pallas import tpu_sc as plsc

N_NODES = 10000
N_EDGES = 320000
D_FEAT = 128
D_EDGE = 16
D_HID = 128
D_OUT = 128

NC = 2
NS = 16
FH = D_EDGE // NC
NFS = 4
FPT = FH // NFS
NGRP = NS // NFS
CBLK = N_EDGES // 128
GBLK = CBLK // NGRP
CB = 125
NCHUNK = GBLK // CB
N_PAD = 10240
NPB = N_PAD // 128


def _seg_sum_sc(ei_v, ea_v4):
    mesh = plsc.VectorSubcoreMesh(core_axis_name="c", subcore_axis_name="s")

    @functools.partial(
        pl.kernel,
        mesh=mesh,
        out_type=jax.ShapeDtypeStruct((NGRP, NC, NPB, FH, 128), jnp.float32),
        scratch_types=[
            pltpu.VMEM((2, CB, 1, 128), jnp.int32),
            pltpu.VMEM((2, CB, FPT, 128), jnp.float32),
            pltpu.VMEM((FPT, NPB, 1, 128), jnp.float32),
            pltpu.SemaphoreType.DMA,
            pltpu.SemaphoreType.DMA,
            pltpu.SemaphoreType.DMA,
            pltpu.SemaphoreType.DMA,
        ],
        compiler_params=pltpu.CompilerParams(
            use_tc_tiling_on_sc=False, needs_layout_passes=False),
    )
    def k(idx_hbm, ea_hbm, out_hbm, idx_v, val_v, agg_v, si0, si1, sv0, sv1):
        cid = lax.axis_index("c")
        sid = lax.axis_index("s")
        fq = sid // NGRP
        grp = sid % NGRP
        cbase = grp * GBLK
        sis = (si0, si1)
        svs = (sv0, sv1)

        def start(j):
            b = j % 2
            c0 = cbase + j * CB
            ci = pltpu.async_copy(
                idx_hbm.at[pl.ds(c0, CB), pl.ds(0, 1)], idx_v.at[b], sis[b])
            cv = pltpu.async_copy(
                ea_hbm.at[cid, pl.ds(c0, CB), pl.ds(fq * FPT, FPT)],
                val_v.at[b], svs[b])
            return ci, cv

        pend = start(0)

        zrow = jnp.zeros((16,), jnp.float32)

        def zb(c, carry):
            for f in range(FPT):
                for l in range(8):
                    agg_v[f, c, 0, pl.ds(l * 16, 16)] = zrow
            return carry

        lax.fori_loop(0, NPB, zb, 0)

        zi = jnp.zeros((16,), jnp.int32)
        for j in range(NCHUNK):
            b = j % 2
            ci, cv = pend
            ci.wait()
            cv.wait()
            if j + 1 < NCHUNK:
                pend = start(j + 1)

            @plsc.parallel_loop(0, CB, unroll=3)
            def step(c):
                for l in range(8):
                    idxv = idx_v[b, c, 0, pl.ds(l * 16, 16)]
                    hi = idxv >> 7
                    lo = idxv & 127
                    for f in range(FPT):
                        valv = val_v[b, c, f, pl.ds(l * 16, 16)]
                        plsc.addupdate_scatter(
                            agg_v.at[f], [hi, zi, lo], valv)

        for f in range(FPT):
            pltpu.sync_copy(
                agg_v.at[f],
                out_hbm.at[grp, cid, pl.ds(0, NPB),
                           pl.ds(fq * FPT + f, 1)])

    return k(ei_v, ea_v4)


ROW_BLK = 2560
N_BLK = N_PAD // ROW_BLK


def _row_spec():
    return pl.BlockSpec((ROW_BLK, D_FEAT), lambda i: (i, 0))


def _full(shape):
    return pl.BlockSpec(shape, lambda i: (0,) * len(shape))


def _dense_body(x_r, xl_r, z_r, wx_r, wl_r, wz_r, b1_r, o_r):
    acc = jnp.dot(x_r[...], wx_r[...], preferred_element_type=jnp.float32)
    acc = acc + jnp.dot(xl_r[...], wl_r[...], preferred_element_type=jnp.float32)
    acc = acc + jnp.dot(z_r[...], wz_r[...], preferred_element_type=jnp.float32)
    o_r[...] = (acc + b1_r[...]).astype(jnp.bfloat16)


def _mlp_dense(x, xl, z, wx, wl, wz, b1):
    return pl.pallas_call(
        _dense_body,
        grid=(N_BLK,),
        in_specs=[
            _row_spec(), _row_spec(), _row_spec(),
            _full((D_FEAT, D_HID)), _full((D_FEAT, D_HID)),
            _full((D_FEAT, D_HID)), _full((1, D_HID)),
        ],
        out_specs=pl.BlockSpec((ROW_BLK, D_HID), lambda i: (i, 0)),
        out_shape=jax.ShapeDtypeStruct((N_NODES, D_HID), jnp.bfloat16),
        compiler_params=pltpu.CompilerParams(
            dimension_semantics=("arbitrary",),
        ),
    )(x, xl, z, wx, wl, wz, b1)


def _final_body(acc_r, parts_r, wa_r, w2_r, b2_r, o_r):
    p = parts_r[...]
    agg_t = p[0] + p[1]
    for i in range(2, NGRP):
        agg_t = agg_t + p[i]
    acc = acc_r[...].astype(jnp.float32) + lax.dot_general(
        agg_t, wa_r[...], (((0,), (0,)), ((), ())),
        preferred_element_type=jnp.float32)
    h = jnp.maximum(acc, 0.0)
    o_r[...] = jnp.dot(h, w2_r[...], preferred_element_type=jnp.float32) + b2_r[...]


def _mlp_final(acc, parts, wa, w2, b2):
    parts_spec = pl.BlockSpec((NGRP, D_EDGE, ROW_BLK), lambda i: (0, 0, i))
    return pl.pallas_call(
        _final_body,
        grid=(N_BLK,),
        in_specs=[
            pl.BlockSpec((ROW_BLK, D_HID), lambda i: (i, 0)), parts_spec,
            _full((D_EDGE, D_HID)), _full((D_HID, D_OUT)), _full((1, D_OUT)),
        ],
        out_specs=pl.BlockSpec((ROW_BLK, D_OUT), lambda i: (i, 0)),
        out_shape=jax.ShapeDtypeStruct((N_NODES, D_OUT), jnp.float32),
        compiler_params=pltpu.CompilerParams(
            dimension_semantics=("arbitrary",),
        ),
    )(acc, parts, wa, w2, b2)


def kernel(x, x_lstm, encoded_z_gnss, edge_index, edge_attr, W1, b1, W2, b2):
    ei = edge_index.astype(jnp.int32)
    ei_v = ei.reshape(2, CBLK, 128).transpose(1, 0, 2)
    ea_v4 = edge_attr.reshape(CBLK, 128, NC, FH).transpose(2, 0, 3, 1)
    parts5 = _seg_sum_sc(ei_v, ea_v4)
    parts = parts5.transpose(0, 1, 3, 2, 4).reshape(NGRP, D_EDGE, N_PAD)
    wx = W1[0:D_FEAT]
    wl = W1[D_FEAT:2 * D_FEAT]
    wz = W1[2 * D_FEAT:3 * D_FEAT]
    wa = W1[3 * D_FEAT:]
    acc = _mlp_dense(x, x_lstm, encoded_z_gnss, wx, wl, wz,
                     b1.reshape(1, D_HID))
    return _mlp_final(acc, parts, wa, W2, b2.reshape(1, D_OUT))


# speedup vs baseline: 1.9643x; 1.0365x over previous
import functools

import jax
import jax.numpy as jnp
from jax import lax
from jax.experimental import pallas as pl
from jax.experimental.pallas import tpu as pltpu
from jax.experimental.pallas import tpu_sc as plsc

N_NODES = 10000
N_EDGES = 320000
D_FEAT = 128
D_EDGE = 16
D_HID = 128
D_OUT = 128

NC = 2
NS = 16
FH = D_EDGE // NC
NFS = 4
FPT = FH // NFS
NGRP = NS // NFS
CBLK = N_EDGES // 128
GBLK = CBLK // NGRP
CB = 125
NCHUNK = GBLK // CB
N_PAD = 10240
NPB = N_PAD // 128


def _seg_sum_sc(ei_v, ea_v4):
    mesh = plsc.VectorSubcoreMesh(core_axis_name="c", subcore_axis_name="s")

    @functools.partial(
        pl.kernel,
        mesh=mesh,
        out_type=jax.ShapeDtypeStruct((NGRP, NC, NPB, FH, 128), jnp.float32),
        scratch_types=[
            pltpu.VMEM((2, CB, 1, 128), jnp.int32),
            pltpu.VMEM((2, CB, FPT, 128), jnp.float32),
            pltpu.VMEM((FPT, NPB, 1, 128), jnp.float32),
            pltpu.SemaphoreType.DMA,
            pltpu.SemaphoreType.DMA,
            pltpu.SemaphoreType.DMA,
            pltpu.SemaphoreType.DMA,
        ],
        compiler_params=pltpu.CompilerParams(
            use_tc_tiling_on_sc=False, needs_layout_passes=False),
    )
    def k(idx_hbm, ea_hbm, out_hbm, idx_v, val_v, agg_v, si0, si1, sv0, sv1):
        cid = lax.axis_index("c")
        sid = lax.axis_index("s")
        fq = sid // NGRP
        grp = sid % NGRP
        cbase = grp * GBLK
        sis = (si0, si1)
        svs = (sv0, sv1)

        def start(j):
            b = j % 2
            c0 = cbase + j * CB
            ci = pltpu.async_copy(
                idx_hbm.at[pl.ds(c0, CB), pl.ds(0, 1)], idx_v.at[b], sis[b])
            cv = pltpu.async_copy(
                ea_hbm.at[cid, pl.ds(c0, CB), pl.ds(fq * FPT, FPT)],
                val_v.at[b], svs[b])
            return ci, cv

        pend = start(0)

        zrow = jnp.zeros((16,), jnp.float32)

        def zb(c, carry):
            for f in range(FPT):
                for l in range(8):
                    agg_v[f, c, 0, pl.ds(l * 16, 16)] = zrow
            return carry

        lax.fori_loop(0, NPB, zb, 0)

        zi = jnp.zeros((16,), jnp.int32)
        for j in range(NCHUNK):
            b = j % 2
            ci, cv = pend
            ci.wait()
            cv.wait()
            if j + 1 < NCHUNK:
                pend = start(j + 1)

            @plsc.parallel_loop(0, CB, unroll=3)
            def step(c):
                for l in range(8):
                    idxv = idx_v[b, c, 0, pl.ds(l * 16, 16)]
                    hi = idxv >> 7
                    lo = idxv & 127
                    for f in range(FPT):
                        valv = val_v[b, c, f, pl.ds(l * 16, 16)]
                        plsc.addupdate_scatter(
                            agg_v.at[f], [hi, zi, lo], valv)

        for f in range(FPT):
            pltpu.sync_copy(
                agg_v.at[f],
                out_hbm.at[grp, cid, pl.ds(0, NPB),
                           pl.ds(fq * FPT + f, 1)])

    return k(ei_v, ea_v4)


ROW_BLK = 5120
N_BLK = N_PAD // ROW_BLK


def _row_spec():
    return pl.BlockSpec((ROW_BLK, D_FEAT), lambda i: (i, 0))


def _full(shape):
    return pl.BlockSpec(shape, lambda i: (0,) * len(shape))


def _dense_body(x_r, xl_r, z_r, wx_r, wl_r, wz_r, b1_r, o_r):
    acc = jnp.dot(x_r[...], wx_r[...], preferred_element_type=jnp.float32)
    acc = acc + jnp.dot(xl_r[...], wl_r[...], preferred_element_type=jnp.float32)
    acc = acc + jnp.dot(z_r[...], wz_r[...], preferred_element_type=jnp.float32)
    o_r[...] = (acc + b1_r[...]).astype(jnp.bfloat16)


def _mlp_dense(x, xl, z, wx, wl, wz, b1):
    return pl.pallas_call(
        _dense_body,
        grid=(N_BLK,),
        in_specs=[
            _row_spec(), _row_spec(), _row_spec(),
            _full((D_FEAT, D_HID)), _full((D_FEAT, D_HID)),
            _full((D_FEAT, D_HID)), _full((1, D_HID)),
        ],
        out_specs=pl.BlockSpec((ROW_BLK, D_HID), lambda i: (i, 0)),
        out_shape=jax.ShapeDtypeStruct((N_NODES, D_HID), jnp.bfloat16),
        compiler_params=pltpu.CompilerParams(
            dimension_semantics=("arbitrary",),
        ),
    )(x, xl, z, wx, wl, wz, b1)


def _final_body(acc_r, parts_r, wa_r, w2_r, b2_r, o_r):
    p = parts_r[...]
    agg_t = p[0] + p[1]
    for i in range(2, NGRP):
        agg_t = agg_t + p[i]
    acc = acc_r[...].astype(jnp.float32) + lax.dot_general(
        agg_t, wa_r[...], (((0,), (0,)), ((), ())),
        preferred_element_type=jnp.float32)
    h = jnp.maximum(acc, 0.0)
    o_r[...] = jnp.dot(h, w2_r[...], preferred_element_type=jnp.float32) + b2_r[...]


def _mlp_final(acc, parts, wa, w2, b2):
    parts_spec = pl.BlockSpec((NGRP, D_EDGE, ROW_BLK), lambda i: (0, 0, i))
    return pl.pallas_call(
        _final_body,
        grid=(N_BLK,),
        in_specs=[
            pl.BlockSpec((ROW_BLK, D_HID), lambda i: (i, 0)), parts_spec,
            _full((D_EDGE, D_HID)), _full((D_HID, D_OUT)), _full((1, D_OUT)),
        ],
        out_specs=pl.BlockSpec((ROW_BLK, D_OUT), lambda i: (i, 0)),
        out_shape=jax.ShapeDtypeStruct((N_NODES, D_OUT), jnp.float32),
        compiler_params=pltpu.CompilerParams(
            dimension_semantics=("arbitrary",),
        ),
    )(acc, parts, wa, w2, b2)


def kernel(x, x_lstm, encoded_z_gnss, edge_index, edge_attr, W1, b1, W2, b2):
    ei = edge_index.astype(jnp.int32)
    ei_v = ei.reshape(2, CBLK, 128).transpose(1, 0, 2)
    ea_v4 = edge_attr.reshape(CBLK, 128, NC, FH).transpose(2, 0, 3, 1)
    parts5 = _seg_sum_sc(ei_v, ea_v4)
    parts = parts5.transpose(0, 1, 3, 2, 4).reshape(NGRP, D_EDGE, N_PAD)
    wx = W1[0:D_FEAT]
    wl = W1[D_FEAT:2 * D_FEAT]
    wz = W1[2 * D_FEAT:3 * D_FEAT]
    wa = W1[3 * D_FEAT:]
    acc = _mlp_dense(x, x_lstm, encoded_z_gnss, wx, wl, wz,
                     b1.reshape(1, D_HID))
    return _mlp_final(acc, parts, wa, W2, b2.reshape(1, D_OUT))
